# baseline (speedup 1.0000x reference)
import functools

import jax
import jax.numpy as jnp
from jax import lax
from jax.experimental import pallas as pl
from jax.experimental.pallas import tpu as pltpu
from jax.experimental.pallas import tpu_sc as plsc

N = 10000
E = 320000
D = 128
C = 1000

NC = 2
NS = 16
NW = NC * NS

N_PAD = 10240
RPT = N_PAD // NS
ROWS_PW = N_PAD // NW

ECH = 128
KE = 80
IBLK = 16
E_PAD = ECH * KE * NW
TOTCH = E_PAD // ECH
KE0 = 160
KE1 = 2 * KE - KE0
NBLK0 = KE0 // IBLK
NBLK1 = KE1 // IBLK

C_PAD = 1024
CRT = C_PAD // NS
CCH = 80
KC = ROWS_PW // CCH


def _sc_mesh():
    return plsc.VectorSubcoreMesh(core_axis_name="c", subcore_axis_name="s")


def _edge_scatter(h, src3, dst3, zeros_n):
    @functools.partial(
        pl.kernel,
        out_type=jax.ShapeDtypeStruct((NC, N_PAD, D), jnp.float32),
        mesh=_sc_mesh(),
        scratch_types=[
            pltpu.VMEM((IBLK, ECH), jnp.int32),
            pltpu.VMEM((IBLK, ECH), jnp.int32),
            pltpu.VMEM((IBLK, ECH), jnp.int32),
            pltpu.VMEM((IBLK, ECH), jnp.int32),
            pltpu.VMEM((ECH, D), jnp.float32),
            pltpu.VMEM((ECH, D), jnp.float32),
            pltpu.VMEM_SHARED((N_PAD, D), jnp.float32),
            pltpu.SemaphoreType.DMA,
            pltpu.SemaphoreType.DMA,
            pltpu.SemaphoreType.DMA,
            pltpu.SemaphoreType.DMA,
        ],
    )
    def k(h_hbm, src_hbm, dst_hbm, z_hbm, out_hbm, sb0, sb1, db0, db1,
          r0, r1, agg_sh, g0, g1, i0, i1):
        rows = [r0, r1]
        semg = [g0, g1]
        sbuf = [sb0, sb1]
        dbuf = [db0, db1]
        semi = [i0, i1]
        cid = lax.axis_index("c")
        sid = lax.axis_index("s")

        def do_block(sb, db):
            for b in range(2):
                pltpu.async_copy(h_hbm.at[sb.at[b]], rows[b], semg[b])

            def rnd(r, carry):
                for b in range(2):
                    c = 2 * r + b
                    pltpu.make_async_copy(
                        h_hbm.at[sb.at[c]], rows[b], semg[b]).wait()
                    pltpu.sync_copy(rows[b], agg_sh.at[db.at[c]], add=True)
                    pltpu.async_copy(h_hbm.at[sb.at[c + 2]], rows[b],
                                     semg[b])
                return carry

            lax.fori_loop(0, IBLK // 2 - 1, rnd, 0)
            for b in range(2):
                c = IBLK - 2 + b
                pltpu.make_async_copy(
                    h_hbm.at[sb.at[c]], rows[b], semg[b]).wait()
                pltpu.sync_copy(rows[b], agg_sh.at[db.at[c]], add=True)

        cb = jnp.where(cid == 0, sid * KE0, 0)
        nblk = jnp.where(cid == 0, NBLK0, NBLK1)

        pltpu.sync_copy(z_hbm.at[pl.ds(sid * RPT, RPT)],
                        agg_sh.at[pl.ds(sid * RPT, RPT)])
        for p in range(2):
            pltpu.async_copy(src_hbm.at[pl.ds(cb + p * IBLK, IBLK)],
                             sbuf[p], semi[p])
            pltpu.async_copy(dst_hbm.at[pl.ds(cb + p * IBLK, IBLK)],
                             dbuf[p], semi[p])
        plsc.subcore_barrier()

        def blkpair(j, carry):
            for p in range(2):
                nb = 2 * j + p
                off = cb + nb * IBLK
                pltpu.make_async_copy(
                    src_hbm.at[pl.ds(off, IBLK)], sbuf[p], semi[p]).wait()
                pltpu.make_async_copy(
                    dst_hbm.at[pl.ds(off, IBLK)], dbuf[p], semi[p]).wait()
                do_block(sbuf[p], dbuf[p])

                @pl.when(nb + 2 < nblk)
                def _():
                    off2 = cb + (nb + 2) * IBLK
                    pltpu.async_copy(src_hbm.at[pl.ds(off2, IBLK)],
                                     sbuf[p], semi[p])
                    pltpu.async_copy(dst_hbm.at[pl.ds(off2, IBLK)],
                                     dbuf[p], semi[p])
            return carry

        lax.fori_loop(0, nblk // 2, blkpair, 0)

        @pl.when(cid == 1)
        def _():
            for p in range(2):
                pltpu.make_async_copy(
                    src_hbm.at[pl.ds(cb + p * IBLK, IBLK)], sbuf[p],
                    semi[p]).wait()
                pltpu.make_async_copy(
                    dst_hbm.at[pl.ds(cb + p * IBLK, IBLK)], dbuf[p],
                    semi[p]).wait()

        plsc.subcore_barrier()
        pltpu.sync_copy(agg_sh.at[pl.ds(sid * RPT, RPT)],
                        out_hbm.at[cid, pl.ds(sid * RPT, RPT)])

    return k(h, src3, dst3, zeros_n)


def _tc_layer(agg, h, Wn, Wr, b8, *, relu, ext):
    BLK = 1024

    def body(a0, a1, hh, wn, wr, bb, oo, of=None):
        acc = jnp.dot(a0[0], wn[...], preferred_element_type=jnp.float32)
        acc = acc + jnp.dot(hh[...], wr[...], preferred_element_type=jnp.float32)
        acc = acc + jnp.dot(a1[0], wn[...], preferred_element_type=jnp.float32)
        acc = acc + bb[0:1, :]
        if relu:
            acc = jnp.maximum(acc, 0.0)
        if ext:
            i = pl.program_id(0)
            rows = i * BLK + lax.broadcasted_iota(jnp.int32, (BLK, 1), 0)
            m = (rows < N).astype(jnp.float32)
            oo[...] = acc * m
            of[...] = jnp.broadcast_to(m, (BLK, D))
        else:
            oo[...] = acc

    if ext:
        out_specs = [pl.BlockSpec((BLK, D), lambda i: (i, 0)),
                     pl.BlockSpec((BLK, D), lambda i: (i, 0))]
        out_shape = [jax.ShapeDtypeStruct((N_PAD, D), jnp.float32),
                     jax.ShapeDtypeStruct((N_PAD, D), jnp.float32)]
    else:
        out_specs = pl.BlockSpec((BLK, D), lambda i: (i, 0))
        out_shape = jax.ShapeDtypeStruct((N_PAD, D), jnp.float32)

    return pl.pallas_call(
        body,
        grid=(N_PAD // BLK,),
        in_specs=[
            pl.BlockSpec((1, BLK, D), lambda i: (0, i, 0)),
            pl.BlockSpec((1, BLK, D), lambda i: (1, i, 0)),
            pl.BlockSpec((BLK, D), lambda i: (i, 0)),
            pl.BlockSpec((D, D), lambda i: (0, 0)),
            pl.BlockSpec((D, D), lambda i: (0, 0)),
            pl.BlockSpec((8, D), lambda i: (0, 0)),
        ],
        out_specs=out_specs,
        out_shape=out_shape,
    )(agg, agg, h, Wn, Wr, b8)


def _cluster_scatter(h2v, h2f, cl3, zeros_c):
    @functools.partial(
        pl.kernel,
        out_type=[jax.ShapeDtypeStruct((NC, C_PAD, D), jnp.float32),
                  jax.ShapeDtypeStruct((NC, C_PAD, D), jnp.float32)],
        mesh=_sc_mesh(),
        scratch_types=[
            pltpu.VMEM((KC, CCH), jnp.int32),
            pltpu.VMEM((CCH, D), jnp.float32),
            pltpu.VMEM((CCH, D), jnp.float32),
            pltpu.VMEM_SHARED((C_PAD, D), jnp.float32),
            pltpu.VMEM_SHARED((C_PAD, D), jnp.float32),
        ],
    )
    def k(hv_hbm, hf_hbm, cl_hbm, z_hbm, outv_hbm, outf_hbm,
          cl_v, rv_v, rf_v, tabv_sh, tabf_sh):
        cid = lax.axis_index("c")
        sid = lax.axis_index("s")
        wid = cid * NS + sid
        pltpu.sync_copy(z_hbm.at[pl.ds(sid * CRT, CRT)],
                        tabv_sh.at[pl.ds(sid * CRT, CRT)])
        pltpu.sync_copy(z_hbm.at[pl.ds(sid * CRT, CRT)],
                        tabf_sh.at[pl.ds(sid * CRT, CRT)])
        pltpu.sync_copy(cl_hbm.at[wid], cl_v)
        plsc.subcore_barrier()
        base = wid * ROWS_PW

        def body(kk, carry):
            pltpu.sync_copy(hv_hbm.at[pl.ds(base + kk * CCH, CCH)], rv_v)
            pltpu.sync_copy(rv_v, tabv_sh.at[cl_v.at[kk]], add=True)
            pltpu.sync_copy(hf_hbm.at[pl.ds(base + kk * CCH, CCH)], rf_v)
            pltpu.sync_copy(rf_v, tabf_sh.at[cl_v.at[kk]], add=True)
            return carry

        lax.fori_loop(0, KC, body, 0)
        plsc.subcore_barrier()
        pltpu.sync_copy(tabv_sh.at[pl.ds(sid * CRT, CRT)],
                        outv_hbm.at[cid, pl.ds(sid * CRT, CRT)])
        pltpu.sync_copy(tabf_sh.at[pl.ds(sid * CRT, CRT)],
                        outf_hbm.at[cid, pl.ds(sid * CRT, CRT)])

    return k(h2v, h2f, cl3, zeros_c)


def _tc_mean(csv, csf):
    def body(v0, v1, f0, f1, oo):
        tot = v0[0] + v1[0]
        cnt = f0[0] + f1[0]
        oo[...] = tot / jnp.maximum(cnt, 1.0)

    spec = lambda j: pl.BlockSpec((1, C_PAD, D), lambda i: (j, 0, 0))
    return pl.pallas_call(
        body,
        grid=(1,),
        in_specs=[spec(0), spec(1), spec(0), spec(1)],
        out_specs=pl.BlockSpec((C_PAD, D), lambda i: (0, 0)),
        out_shape=jax.ShapeDtypeStruct((C_PAD, D), jnp.float32),
    )(csv, csv, csf, csf)


def _cluster_gather(mean, cl3):
    @functools.partial(
        pl.kernel,
        out_type=jax.ShapeDtypeStruct((N_PAD, D), jnp.float32),
        mesh=_sc_mesh(),
        scratch_types=[
            pltpu.VMEM((KC, CCH), jnp.int32),
            pltpu.VMEM((CCH, D), jnp.float32),
            pltpu.SemaphoreType.DMA,
        ],
    )
    def k(mean_hbm, cl_hbm, out_hbm, cl_v, rows_v, sem):
        cid = lax.axis_index("c")
        sid = lax.axis_index("s")
        wid = cid * NS + sid
        pltpu.sync_copy(cl_hbm.at[wid], cl_v)
        base = wid * ROWS_PW

        def body(kk, carry):
            pltpu.async_copy(mean_hbm.at[cl_v.at[kk]], rows_v, sem).wait()
            pltpu.sync_copy(rows_v, out_hbm.at[pl.ds(base + kk * CCH, CCH)])
            return carry

        lax.fori_loop(0, KC, body, 0)

    return k(mean, cl3)


def kernel(x, edge_index, cluster_assignment, W1n, W1r, b1, W2n, W2r, b2):
    src = edge_index[0]
    dst = edge_index[1]
    pad_e = E_PAD - E
    src3 = jnp.concatenate(
        [src, jnp.zeros((pad_e,), jnp.int32)]).reshape(TOTCH, ECH)
    dst3 = jnp.concatenate(
        [dst, jnp.full((pad_e,), N_PAD - 1, jnp.int32)]).reshape(TOTCH, ECH)
    cl3 = jnp.pad(cluster_assignment, (0, N_PAD - N)).reshape(NW, KC, CCH)
    x_pad = jnp.pad(x, ((0, N_PAD - N), (0, 0)))
    zeros_n = jnp.zeros((N_PAD, D), jnp.float32)
    zeros_c = jnp.zeros((C_PAD, D), jnp.float32)
    b1b = jnp.broadcast_to(b1.reshape(1, D), (8, D))
    b2b = jnp.broadcast_to(b2.reshape(1, D), (8, D))

    agg1 = _edge_scatter(x_pad, src3, dst3, zeros_n)
    h1 = _tc_layer(agg1, x_pad, W1n, W1r, b1b, relu=True, ext=False)
    agg2 = _edge_scatter(h1, src3, dst3, zeros_n)
    h2v, h2f = _tc_layer(agg2, h1, W2n, W2r, b2b, relu=False, ext=True)
    csv, csf = _cluster_scatter(h2v, h2f, cl3, zeros_c)
    mean = _tc_mean(csv, csf)
    outp = _cluster_gather(mean, cl3)
    return outp[:N]


# speedup vs baseline: 1.2660x; 1.2660x over previous
import functools

import jax
import jax.numpy as jnp
from jax import lax
from jax.experimental import pallas as pl
from jax.experimental.pallas import tpu as pltpu
from jax.experimental.pallas import tpu_sc as plsc

N = 10000
E = 320000
D = 128
C = 1000

NC = 2
NS = 16
NW = NC * NS

N_PAD = 10240
RPT = N_PAD // NS
ROWS_PW = N_PAD // NW

ECH = 128
KE = 80
IBLK = 16
E_PAD = ECH * KE * NW
TOTCH = E_PAD // ECH
KE0 = 128
KE1 = 2 * KE - KE0
NBLK0 = KE0 // IBLK
NBLK1 = KE1 // IBLK

C_PAD = 1024
CRT = C_PAD // NS
CCH = 80
KC = ROWS_PW // CCH


def _sc_mesh():
    return plsc.VectorSubcoreMesh(core_axis_name="c", subcore_axis_name="s")


def _edge_scatter(h, src3, dst3, zeros_n):
    @functools.partial(
        pl.kernel,
        out_type=jax.ShapeDtypeStruct((NC, N_PAD, D), jnp.float32),
        mesh=_sc_mesh(),
        scratch_types=[
            pltpu.VMEM((IBLK, ECH), jnp.int32),
            pltpu.VMEM((IBLK, ECH), jnp.int32),
            pltpu.VMEM((IBLK, ECH), jnp.int32),
            pltpu.VMEM((IBLK, ECH), jnp.int32),
            pltpu.VMEM((ECH, D), jnp.float32),
            pltpu.VMEM((ECH, D), jnp.float32),
            pltpu.VMEM_SHARED((N_PAD, D), jnp.float32),
            pltpu.SemaphoreType.DMA,
            pltpu.SemaphoreType.DMA,
            pltpu.SemaphoreType.DMA,
            pltpu.SemaphoreType.DMA,
        ],
    )
    def k(h_hbm, src_hbm, dst_hbm, z_hbm, out_hbm, sb0, sb1, db0, db1,
          r0, r1, agg_sh, g0, g1, i0, i1):
        rows = [r0, r1]
        semg = [g0, g1]
        sbuf = [sb0, sb1]
        dbuf = [db0, db1]
        semi = [i0, i1]
        cid = lax.axis_index("c")
        sid = lax.axis_index("s")

        def do_block(sb, db):
            for b in range(2):
                pltpu.async_copy(h_hbm.at[sb.at[b]], rows[b], semg[b])

            def rnd(r, carry):
                for b in range(2):
                    c = 2 * r + b
                    pltpu.make_async_copy(
                        h_hbm.at[sb.at[c]], rows[b], semg[b]).wait()
                    pltpu.sync_copy(rows[b], agg_sh.at[db.at[c]], add=True)
                    pltpu.async_copy(h_hbm.at[sb.at[c + 2]], rows[b],
                                     semg[b])
                return carry

            lax.fori_loop(0, IBLK // 2 - 1, rnd, 0)
            for b in range(2):
                c = IBLK - 2 + b
                pltpu.make_async_copy(
                    h_hbm.at[sb.at[c]], rows[b], semg[b]).wait()
                pltpu.sync_copy(rows[b], agg_sh.at[db.at[c]], add=True)

        cb = jnp.where(cid == 0, sid * KE0, NS * KE0 + sid * KE1)
        nblk = jnp.where(cid == 0, NBLK0, NBLK1)

        pltpu.sync_copy(z_hbm.at[pl.ds(sid * RPT, RPT)],
                        agg_sh.at[pl.ds(sid * RPT, RPT)])
        for p in range(2):
            pltpu.async_copy(src_hbm.at[pl.ds(cb + p * IBLK, IBLK)],
                             sbuf[p], semi[p])
            pltpu.async_copy(dst_hbm.at[pl.ds(cb + p * IBLK, IBLK)],
                             dbuf[p], semi[p])
        plsc.subcore_barrier()

        def blkpair(j, carry):
            for p in range(2):
                nb = 2 * j + p
                off = cb + nb * IBLK
                pltpu.make_async_copy(
                    src_hbm.at[pl.ds(off, IBLK)], sbuf[p], semi[p]).wait()
                pltpu.make_async_copy(
                    dst_hbm.at[pl.ds(off, IBLK)], dbuf[p], semi[p]).wait()
                do_block(sbuf[p], dbuf[p])

                @pl.when(nb + 2 < nblk)
                def _():
                    off2 = cb + (nb + 2) * IBLK
                    pltpu.async_copy(src_hbm.at[pl.ds(off2, IBLK)],
                                     sbuf[p], semi[p])
                    pltpu.async_copy(dst_hbm.at[pl.ds(off2, IBLK)],
                                     dbuf[p], semi[p])
            return carry

        lax.fori_loop(0, nblk // 2, blkpair, 0)
        plsc.subcore_barrier()
        pltpu.sync_copy(agg_sh.at[pl.ds(sid * RPT, RPT)],
                        out_hbm.at[cid, pl.ds(sid * RPT, RPT)])

    return k(h, src3, dst3, zeros_n)


def _tc_layer(agg, h, Wn, Wr, b8, *, relu, ext):
    BLK = 1024

    def body(a0, a1, hh, wn, wr, bb, oo, of=None):
        acc = jnp.dot(a0[0], wn[...], preferred_element_type=jnp.float32)
        acc = acc + jnp.dot(hh[...], wr[...], preferred_element_type=jnp.float32)
        acc = acc + jnp.dot(a1[0], wn[...], preferred_element_type=jnp.float32)
        acc = acc + bb[0:1, :]
        if relu:
            acc = jnp.maximum(acc, 0.0)
        if ext:
            i = pl.program_id(0)
            rows = i * BLK + lax.broadcasted_iota(jnp.int32, (BLK, 1), 0)
            m = (rows < N).astype(jnp.float32)
            oo[...] = acc * m
            of[...] = jnp.broadcast_to(m, (BLK, D))
        else:
            oo[...] = acc

    if ext:
        out_specs = [pl.BlockSpec((BLK, D), lambda i: (i, 0)),
                     pl.BlockSpec((BLK, D), lambda i: (i, 0))]
        out_shape = [jax.ShapeDtypeStruct((N_PAD, D), jnp.float32),
                     jax.ShapeDtypeStruct((N_PAD, D), jnp.float32)]
    else:
        out_specs = pl.BlockSpec((BLK, D), lambda i: (i, 0))
        out_shape = jax.ShapeDtypeStruct((N_PAD, D), jnp.float32)

    return pl.pallas_call(
        body,
        grid=(N_PAD // BLK,),
        in_specs=[
            pl.BlockSpec((1, BLK, D), lambda i: (0, i, 0)),
            pl.BlockSpec((1, BLK, D), lambda i: (1, i, 0)),
            pl.BlockSpec((BLK, D), lambda i: (i, 0)),
            pl.BlockSpec((D, D), lambda i: (0, 0)),
            pl.BlockSpec((D, D), lambda i: (0, 0)),
            pl.BlockSpec((8, D), lambda i: (0, 0)),
        ],
        out_specs=out_specs,
        out_shape=out_shape,
    )(agg, agg, h, Wn, Wr, b8)


def _cluster_scatter(h2v, h2f, cl3, zeros_c):
    @functools.partial(
        pl.kernel,
        out_type=[jax.ShapeDtypeStruct((NC, C_PAD, D), jnp.float32),
                  jax.ShapeDtypeStruct((NC, C_PAD, D), jnp.float32)],
        mesh=_sc_mesh(),
        scratch_types=[
            pltpu.VMEM((KC, CCH), jnp.int32),
            pltpu.VMEM((CCH, D), jnp.float32),
            pltpu.VMEM((CCH, D), jnp.float32),
            pltpu.VMEM_SHARED((C_PAD, D), jnp.float32),
            pltpu.VMEM_SHARED((C_PAD, D), jnp.float32),
        ],
    )
    def k(hv_hbm, hf_hbm, cl_hbm, z_hbm, outv_hbm, outf_hbm,
          cl_v, rv_v, rf_v, tabv_sh, tabf_sh):
        cid = lax.axis_index("c")
        sid = lax.axis_index("s")
        wid = cid * NS + sid
        pltpu.sync_copy(z_hbm.at[pl.ds(sid * CRT, CRT)],
                        tabv_sh.at[pl.ds(sid * CRT, CRT)])
        pltpu.sync_copy(z_hbm.at[pl.ds(sid * CRT, CRT)],
                        tabf_sh.at[pl.ds(sid * CRT, CRT)])
        pltpu.sync_copy(cl_hbm.at[wid], cl_v)
        plsc.subcore_barrier()
        base = wid * ROWS_PW

        def body(kk, carry):
            pltpu.sync_copy(hv_hbm.at[pl.ds(base + kk * CCH, CCH)], rv_v)
            pltpu.sync_copy(rv_v, tabv_sh.at[cl_v.at[kk]], add=True)
            pltpu.sync_copy(hf_hbm.at[pl.ds(base + kk * CCH, CCH)], rf_v)
            pltpu.sync_copy(rf_v, tabf_sh.at[cl_v.at[kk]], add=True)
            return carry

        lax.fori_loop(0, KC, body, 0)
        plsc.subcore_barrier()
        pltpu.sync_copy(tabv_sh.at[pl.ds(sid * CRT, CRT)],
                        outv_hbm.at[cid, pl.ds(sid * CRT, CRT)])
        pltpu.sync_copy(tabf_sh.at[pl.ds(sid * CRT, CRT)],
                        outf_hbm.at[cid, pl.ds(sid * CRT, CRT)])

    return k(h2v, h2f, cl3, zeros_c)


def _tc_mean(csv, csf):
    def body(v0, v1, f0, f1, oo):
        tot = v0[0] + v1[0]
        cnt = f0[0] + f1[0]
        oo[...] = tot / jnp.maximum(cnt, 1.0)

    spec = lambda j: pl.BlockSpec((1, C_PAD, D), lambda i: (j, 0, 0))
    return pl.pallas_call(
        body,
        grid=(1,),
        in_specs=[spec(0), spec(1), spec(0), spec(1)],
        out_specs=pl.BlockSpec((C_PAD, D), lambda i: (0, 0)),
        out_shape=jax.ShapeDtypeStruct((C_PAD, D), jnp.float32),
    )(csv, csv, csf, csf)


def _cluster_gather(mean, cl3):
    @functools.partial(
        pl.kernel,
        out_type=jax.ShapeDtypeStruct((N_PAD, D), jnp.float32),
        mesh=_sc_mesh(),
        scratch_types=[
            pltpu.VMEM((KC, CCH), jnp.int32),
            pltpu.VMEM((CCH, D), jnp.float32),
            pltpu.SemaphoreType.DMA,
        ],
    )
    def k(mean_hbm, cl_hbm, out_hbm, cl_v, rows_v, sem):
        cid = lax.axis_index("c")
        sid = lax.axis_index("s")
        wid = cid * NS + sid
        pltpu.sync_copy(cl_hbm.at[wid], cl_v)
        base = wid * ROWS_PW

        def body(kk, carry):
            pltpu.async_copy(mean_hbm.at[cl_v.at[kk]], rows_v, sem).wait()
            pltpu.sync_copy(rows_v, out_hbm.at[pl.ds(base + kk * CCH, CCH)])
            return carry

        lax.fori_loop(0, KC, body, 0)

    return k(mean, cl3)


def kernel(x, edge_index, cluster_assignment, W1n, W1r, b1, W2n, W2r, b2):
    src = edge_index[0]
    dst = edge_index[1]
    pad_e = E_PAD - E
    src3 = jnp.concatenate(
        [src, jnp.zeros((pad_e,), jnp.int32)]).reshape(TOTCH, ECH)
    dst3 = jnp.concatenate(
        [dst, jnp.full((pad_e,), N_PAD - 1, jnp.int32)]).reshape(TOTCH, ECH)
    cl3 = jnp.pad(cluster_assignment, (0, N_PAD - N)).reshape(NW, KC, CCH)
    x_pad = jnp.pad(x, ((0, N_PAD - N), (0, 0)))
    zeros_n = jnp.zeros((N_PAD, D), jnp.float32)
    zeros_c = jnp.zeros((C_PAD, D), jnp.float32)
    b1b = jnp.broadcast_to(b1.reshape(1, D), (8, D))
    b2b = jnp.broadcast_to(b2.reshape(1, D), (8, D))

    agg1 = _edge_scatter(x_pad, src3, dst3, zeros_n)
    h1 = _tc_layer(agg1, x_pad, W1n, W1r, b1b, relu=True, ext=False)
    agg2 = _edge_scatter(h1, src3, dst3, zeros_n)
    h2v, h2f = _tc_layer(agg2, h1, W2n, W2r, b2b, relu=False, ext=True)
    csv, csf = _cluster_scatter(h2v, h2f, cl3, zeros_c)
    mean = _tc_mean(csv, csf)
    outp = _cluster_gather(mean, cl3)
    return outp[:N]
